# baseline (speedup 1.0000x reference)
import functools

import jax
import jax.numpy as jnp
from jax.experimental import pallas as pl
from jax.experimental.pallas import tpu as pltpu

CHUNK = 1024
NBUF = 4
NSUB = 4
SUB = CHUNK // NSUB


def _route_chunk(x, w, b):
    P = w.shape[0]
    logits_t = jax.lax.dot_general(
        w, x, (((1,), (1,)), ((), ())),
        preferred_element_type=jnp.float32)
    logits_t = logits_t + b
    m = jnp.max(logits_t, axis=0, keepdims=True)
    sub = jax.lax.broadcasted_iota(jnp.int32, logits_t.shape, 0)
    sel = jnp.min(jnp.where(logits_t == m, sub, P), axis=0)
    sel = sel.astype(jnp.int32)
    mask_t = (sub == sel[None, :]).astype(jnp.float32)
    return sel, mask_t.T


def _router_kernel(x_hbm, w_ref, b_ref, sel_hbm, mask_hbm,
                   xbuf, selbuf, maskbuf, in_sems, sel_sems, mask_sems):
    n_chunks = x_hbm.shape[0] // CHUNK

    def in_copies(c, slot):
        return [pltpu.make_async_copy(
            x_hbm.at[pl.ds(c * CHUNK + k * SUB, SUB), :],
            xbuf.at[slot, pl.ds(k * SUB, SUB), :],
            in_sems.at[slot]) for k in range(NSUB)]

    def mask_copy(c, slot):
        return pltpu.make_async_copy(
            maskbuf.at[slot], mask_hbm.at[pl.ds(c * CHUNK, CHUNK), :],
            mask_sems.at[slot])

    def sel_copy(c, slot):
        return pltpu.make_async_copy(
            selbuf.at[slot], sel_hbm.at[:, pl.ds(c * CHUNK, CHUNK)],
            sel_sems.at[slot])

    for i in range(NBUF):
        for cp in in_copies(i, i):
            cp.start()

    for c in range(n_chunks):
        slot = c % NBUF
        for cp in in_copies(c, slot):
            cp.wait()
        if c >= NBUF:
            mask_copy(c - NBUF, slot).wait()
            sel_copy(c - NBUF, slot).wait()
        sel, mask = _route_chunk(xbuf[slot], w_ref[...], b_ref[...])
        maskbuf[slot] = mask
        selbuf[slot, 0, :] = sel
        mask_copy(c, slot).start()
        sel_copy(c, slot).start()
        if c + NBUF < n_chunks:
            for cp in in_copies(c + NBUF, slot):
                cp.start()

    for c in range(n_chunks - NBUF, n_chunks):
        slot = c % NBUF
        mask_copy(c, slot).wait()
        sel_copy(c, slot).wait()


@functools.partial(jax.jit, static_argnames=())
def kernel(process_feats, routing_matrix, bias):
    B, N, D = process_feats.shape
    P = routing_matrix.shape[0]
    T = B * N
    x = process_feats.reshape(T, D)
    b = bias.reshape(P, 1)
    sel2d, mask = pl.pallas_call(
        _router_kernel,
        in_specs=[
            pl.BlockSpec(memory_space=pltpu.MemorySpace.HBM),
            pl.BlockSpec((P, D), lambda: (0, 0)),
            pl.BlockSpec((P, 1), lambda: (0, 0)),
        ],
        out_specs=[
            pl.BlockSpec(memory_space=pltpu.MemorySpace.HBM),
            pl.BlockSpec(memory_space=pltpu.MemorySpace.HBM),
        ],
        out_shape=[
            jax.ShapeDtypeStruct((1, T), jnp.int32),
            jax.ShapeDtypeStruct((T, P), jnp.float32),
        ],
        scratch_shapes=[
            pltpu.VMEM((NBUF, CHUNK, D), jnp.float32),
            pltpu.VMEM((NBUF, 1, CHUNK), jnp.int32),
            pltpu.VMEM((NBUF, CHUNK, P), jnp.float32),
            pltpu.SemaphoreType.DMA((NBUF,)),
            pltpu.SemaphoreType.DMA((NBUF,)),
            pltpu.SemaphoreType.DMA((NBUF,)),
        ],
    )(x, routing_matrix, b)
    selected = sel2d.reshape(B, N)
    policy_mask = mask.reshape(B, N, P)
    return (selected, policy_mask)


# speedup vs baseline: 1.0219x; 1.0219x over previous
import functools

import jax
import jax.numpy as jnp
from jax.experimental import pallas as pl
from jax.experimental.pallas import tpu as pltpu

BLK = 512
NBUF = 8


def _route_block(x_ref, sel_ref, mask_ref, *, w_ref, b_ref):
    x = x_ref[...]
    w = w_ref[...]
    P = w.shape[0]
    logits_t = jax.lax.dot_general(
        w, x, (((1,), (1,)), ((), ())),
        preferred_element_type=jnp.float32)
    logits_t = logits_t + b_ref[...]
    m = jnp.max(logits_t, axis=0, keepdims=True)
    sub = jax.lax.broadcasted_iota(jnp.int32, logits_t.shape, 0)
    sel = jnp.min(jnp.where(logits_t == m, sub, P), axis=0)
    sel = sel.astype(jnp.int32)
    mask_t = (sub == sel[None, :]).astype(jnp.float32)
    mask_ref[...] = mask_t.T
    sel_ref[0, 0, :] = sel


def _router_kernel(x_hbm, w_ref, b_ref, sel_hbm, mask_hbm):
    T, D = x_hbm.shape
    P = w_ref.shape[0]
    pipeline = pltpu.emit_pipeline(
        functools.partial(_route_block, w_ref=w_ref, b_ref=b_ref),
        grid=(T // BLK,),
        in_specs=[
            pl.BlockSpec((BLK, D), lambda i: (i, 0),
                         pipeline_mode=pl.Buffered(buffer_count=NBUF,
                                                   use_lookahead=True)),
        ],
        out_specs=[
            pl.BlockSpec((1, 1, BLK), lambda i: (i, 0, 0)),
            pl.BlockSpec((BLK, P), lambda i: (i, 0)),
        ],
    )
    pipeline(x_hbm, sel_hbm, mask_hbm)


@functools.partial(jax.jit, static_argnames=())
def kernel(process_feats, routing_matrix, bias):
    B, N, D = process_feats.shape
    P = routing_matrix.shape[0]
    T = B * N
    x = process_feats.reshape(T, D)
    b = bias.reshape(P, 1)
    sel2d, mask = pl.pallas_call(
        _router_kernel,
        in_specs=[
            pl.BlockSpec(memory_space=pltpu.MemorySpace.HBM),
            pl.BlockSpec((P, D), lambda: (0, 0)),
            pl.BlockSpec((P, 1), lambda: (0, 0)),
        ],
        out_specs=[
            pl.BlockSpec(memory_space=pltpu.MemorySpace.HBM),
            pl.BlockSpec(memory_space=pltpu.MemorySpace.HBM),
        ],
        out_shape=[
            jax.ShapeDtypeStruct((T // BLK, 1, BLK), jnp.int32),
            jax.ShapeDtypeStruct((T, P), jnp.float32),
        ],
    )(x, routing_matrix, b)
    selected = sel2d.reshape(B, N)
    policy_mask = mask.reshape(B, N, P)
    return (selected, policy_mask)
